# baseline (speedup 1.0000x reference)
import functools

import jax
import jax.numpy as jnp
from jax import lax
from jax.experimental import pallas as pl
from jax.experimental.pallas import tpu as pltpu
from jax.experimental.pallas import tpu_sc as plsc

EMB = 64
EPS = 1e-5
LANES = 16
VPR = EMB // LANES
CHUNK = 128
NC, NS = 2, 16
NW = NC * NS


def _rsqrt(v):
    bits = lax.bitcast_convert_type(v, jnp.int32)
    seed = jnp.int32(0x5F3759DF) - lax.shift_right_logical(bits, 1)
    y = lax.bitcast_convert_type(seed, jnp.float32)
    for _ in range(3):
        y = y * (1.5 - 0.5 * v * y * y)
    return y


def kernel(input_tensor, table, gamma, beta):
    B, L = input_tensor.shape
    N = B * L
    per_w = N // NW
    n_chunks = per_w // CHUNK
    idx = input_tensor.reshape(NW, n_chunks, CHUNK).astype(jnp.int32)

    mesh = plsc.VectorSubcoreMesh(core_axis_name="c", subcore_axis_name="s")

    @functools.partial(
        pl.kernel,
        out_type=jax.ShapeDtypeStruct((N, EMB), jnp.float32),
        mesh=mesh,
        compiler_params=pltpu.CompilerParams(
            needs_layout_passes=False, use_tc_tiling_on_sc=False),
        scratch_types=[
            pltpu.VMEM((n_chunks, CHUNK), jnp.int32),
            pltpu.VMEM((CHUNK, EMB), jnp.float32),
            pltpu.VMEM((CHUNK, EMB), jnp.float32),
            pltpu.VMEM((EMB,), jnp.float32),
            pltpu.VMEM((EMB,), jnp.float32),
            pltpu.SemaphoreType.DMA,
        ],
    )
    def sc_kernel(idx_hbm, table_hbm, gamma_hbm, beta_hbm, out_hbm,
                  idx_v, rows_v, out_v, gamma_v, beta_v, sem):
        wid = lax.axis_index("s") * NC + lax.axis_index("c")
        pltpu.sync_copy(idx_hbm.at[wid], idx_v)
        pltpu.sync_copy(gamma_hbm, gamma_v)
        pltpu.sync_copy(beta_hbm, beta_v)
        g_vecs = [gamma_v[pl.ds(i * LANES, LANES)] for i in range(VPR)]
        b_vecs = [beta_v[pl.ds(i * LANES, LANES)] for i in range(VPR)]
        base = wid * per_w

        def chunk_body(g, _):
            pltpu.async_copy(table_hbm.at[idx_v.at[g]], rows_v, sem).wait()

            def row_body(r, _):
                vs = [rows_v[r, pl.ds(i * LANES, LANES)] for i in range(VPR)]
                tot = jnp.sum(vs[0] + vs[1] + vs[2] + vs[3])
                totsq = jnp.sum(vs[0] * vs[0] + vs[1] * vs[1]
                                + vs[2] * vs[2] + vs[3] * vs[3])
                mean = tot * (1.0 / EMB)
                var = jnp.maximum(totsq * (1.0 / EMB) - mean * mean, 0.0) + EPS
                mean_b = jnp.full((LANES,), mean, jnp.float32)
                rstd_b = _rsqrt(jnp.full((LANES,), var, jnp.float32))
                for i in range(VPR):
                    out_v[r, pl.ds(i * LANES, LANES)] = (
                        (vs[i] - mean_b) * rstd_b * g_vecs[i] + b_vecs[i])
                return 0

            lax.fori_loop(0, CHUNK, row_body, 0)
            pltpu.sync_copy(out_v, out_hbm.at[pl.ds(base + g * CHUNK, CHUNK)])
            return 0

        lax.fori_loop(0, n_chunks, chunk_body, 0)

    out = sc_kernel(idx, table, gamma, beta)
    return out.reshape(B, L, EMB)


# speedup vs baseline: 1.2237x; 1.2237x over previous
import functools

import jax
import jax.numpy as jnp
from jax import lax
from jax.experimental import pallas as pl
from jax.experimental.pallas import tpu as pltpu
from jax.experimental.pallas import tpu_sc as plsc

EMB = 64
EPS = 1e-5
LANES = 16
VPR = EMB // LANES
STREAM = 128
SUB = 2
BLOCK = STREAM * SUB
NBUF = 2
RUNROLL = 4
NC, NS = 2, 16
NW = NC * NS


def _rsqrt(v):
    bits = lax.bitcast_convert_type(v, jnp.int32)
    seed = jnp.int32(0x5F3759DF) - lax.shift_right_logical(bits, 1)
    y = lax.bitcast_convert_type(seed, jnp.float32)
    for _ in range(3):
        y = y * (1.5 - 0.5 * v * y * y)
    return y


def kernel(input_tensor, table, gamma, beta):
    B, L = input_tensor.shape
    N = B * L
    per_w = N // NW
    n_blocks = per_w // BLOCK
    n_outer = n_blocks // NBUF
    idx = input_tensor.reshape(NW, n_blocks, SUB, STREAM).astype(jnp.int32)

    mesh = plsc.VectorSubcoreMesh(core_axis_name="c", subcore_axis_name="s")

    @functools.partial(
        pl.kernel,
        out_type=jax.ShapeDtypeStruct((N, EMB), jnp.float32),
        mesh=mesh,
        compiler_params=pltpu.CompilerParams(
            needs_layout_passes=False, use_tc_tiling_on_sc=False),
        scratch_types=[
            pltpu.VMEM((n_blocks, SUB, STREAM), jnp.int32),
            [pltpu.VMEM((BLOCK, EMB), jnp.float32) for _ in range(NBUF)],
            [pltpu.VMEM((BLOCK, EMB), jnp.float32) for _ in range(NBUF)],
            pltpu.VMEM((EMB,), jnp.float32),
            pltpu.VMEM((EMB,), jnp.float32),
            [pltpu.SemaphoreType.DMA for _ in range(NBUF)],
            [pltpu.SemaphoreType.DMA for _ in range(NBUF)],
        ],
    )
    def sc_kernel(idx_hbm, table_hbm, gamma_hbm, beta_hbm, out_hbm,
                  idx_v, rows_v, out_v, gamma_v, beta_v, gsem, ssem):
        wid = lax.axis_index("s") * NC + lax.axis_index("c")
        pltpu.sync_copy(idx_hbm.at[wid], idx_v)
        pltpu.sync_copy(gamma_hbm, gamma_v)
        pltpu.sync_copy(beta_hbm, beta_v)
        g_vecs = [gamma_v[pl.ds(i * LANES, LANES)] for i in range(VPR)]
        b_vecs = [beta_v[pl.ds(i * LANES, LANES)] for i in range(VPR)]
        base = wid * per_w

        def start_gather(g, b):
            for j in range(SUB):
                pltpu.async_copy(
                    table_hbm.at[idx_v.at[g, j]],
                    rows_v[b].at[pl.ds(j * STREAM, STREAM)], gsem[b])

        def wait_gather(b):
            for j in range(SUB):
                pltpu.make_async_copy(
                    table_hbm.at[idx_v.at[0, j]],
                    rows_v[b].at[pl.ds(j * STREAM, STREAM)], gsem[b]).wait()

        def compute_block(b):
            rows, out = rows_v[b], out_v[b]

            def row_body(r0, _):
                for u in range(RUNROLL):
                    r = r0 * RUNROLL + u
                    vs = [rows[r, pl.ds(i * LANES, LANES)] for i in range(VPR)]
                    tot = jnp.sum(vs[0] + vs[1] + vs[2] + vs[3])
                    totsq = jnp.sum(vs[0] * vs[0] + vs[1] * vs[1]
                                    + vs[2] * vs[2] + vs[3] * vs[3])
                    mean = tot * (1.0 / EMB)
                    var = jnp.maximum(
                        totsq * (1.0 / EMB) - mean * mean, 0.0) + EPS
                    mean_b = jnp.full((LANES,), mean, jnp.float32)
                    rstd_b = _rsqrt(jnp.full((LANES,), var, jnp.float32))
                    for i in range(VPR):
                        out[r, pl.ds(i * LANES, LANES)] = (
                            (vs[i] - mean_b) * rstd_b * g_vecs[i] + b_vecs[i])
                return 0

            lax.fori_loop(0, BLOCK // RUNROLL, row_body, 0)

        def store_wait(b):
            pltpu.make_async_copy(
                out_v[b], out_hbm.at[pl.ds(base, BLOCK)], ssem[b]).wait()

        for b in range(NBUF):
            start_gather(b, b)

        def outer(t, _):
            for b in range(NBUF):
                g = t * NBUF + b
                wait_gather(b)

                @pl.when(t >= 1)
                def _():
                    store_wait(b)

                compute_block(b)
                pltpu.async_copy(
                    out_v[b], out_hbm.at[pl.ds(base + g * BLOCK, BLOCK)],
                    ssem[b])

                @pl.when(t < n_outer - 1)
                def _():
                    start_gather(g + NBUF, b)
            return 0

        lax.fori_loop(0, n_outer, outer, 0)
        for b in range(NBUF):
            store_wait(b)

    out = sc_kernel(idx, table, gamma, beta)
    return out.reshape(B, L, EMB)


# speedup vs baseline: 1.2254x; 1.0014x over previous
import functools

import jax
import jax.numpy as jnp
from jax import lax
from jax.experimental import pallas as pl
from jax.experimental.pallas import tpu as pltpu
from jax.experimental.pallas import tpu_sc as plsc

EMB = 64
EPS = 1e-5
LANES = 16
VPR = EMB // LANES
NBUF = 2
RUNROLL = 4
NC, NS = 2, 16
NW = NC * NS


def _rsqrt(v):
    bits = lax.bitcast_convert_type(v, jnp.int32)
    seed = jnp.int32(0x5F3759DF) - lax.shift_right_logical(bits, 1)
    y = lax.bitcast_convert_type(seed, jnp.float32)
    for _ in range(3):
        y = y * (1.5 - 0.5 * v * y * y)
    return y


def kernel(input_tensor, table, gamma, beta):
    B, L = input_tensor.shape
    rows_per_w = B // NW
    n_outer = rows_per_w // NBUF
    splits = []
    off = 0
    while off < L:
        n = min(128, L - off)
        splits.append((off, n))
        off += n
    idx = input_tensor.astype(jnp.int32)

    mesh = plsc.VectorSubcoreMesh(core_axis_name="c", subcore_axis_name="s")

    @functools.partial(
        pl.kernel,
        out_type=jax.ShapeDtypeStruct((B, L, EMB), jnp.float32),
        mesh=mesh,
        compiler_params=pltpu.CompilerParams(
            needs_layout_passes=False, use_tc_tiling_on_sc=False),
        scratch_types=[
            pltpu.VMEM((rows_per_w, L), jnp.int32),
            [pltpu.VMEM((L, EMB), jnp.float32) for _ in range(NBUF)],
            [pltpu.VMEM((L, EMB), jnp.float32) for _ in range(NBUF)],
            pltpu.VMEM((EMB,), jnp.float32),
            pltpu.VMEM((EMB,), jnp.float32),
            [pltpu.SemaphoreType.DMA for _ in range(NBUF)],
            [pltpu.SemaphoreType.DMA for _ in range(NBUF)],
        ],
    )
    def sc_kernel(idx_hbm, table_hbm, gamma_hbm, beta_hbm, out_hbm,
                  idx_v, rows_v, out_v, gamma_v, beta_v, gsem, ssem):
        wid = lax.axis_index("s") * NC + lax.axis_index("c")
        base = wid * rows_per_w
        pltpu.sync_copy(idx_hbm.at[pl.ds(base, rows_per_w)], idx_v)
        pltpu.sync_copy(gamma_hbm, gamma_v)
        pltpu.sync_copy(beta_hbm, beta_v)
        g_vecs = [gamma_v[pl.ds(i * LANES, LANES)] for i in range(VPR)]
        b_vecs = [beta_v[pl.ds(i * LANES, LANES)] for i in range(VPR)]

        def start_gather(r, b):
            for off, n in splits:
                pltpu.async_copy(
                    table_hbm.at[idx_v.at[r, pl.ds(off, n)]],
                    rows_v[b].at[pl.ds(off, n)], gsem[b])

        def wait_gather(b):
            for off, n in splits:
                pltpu.make_async_copy(
                    table_hbm.at[idx_v.at[0, pl.ds(off, n)]],
                    rows_v[b].at[pl.ds(off, n)], gsem[b]).wait()

        def compute_block(b):
            rows, out = rows_v[b], out_v[b]

            def row_body(r0, _):
                for u in range(RUNROLL):
                    r = r0 * RUNROLL + u
                    vs = [rows[r, pl.ds(i * LANES, LANES)] for i in range(VPR)]
                    tot = jnp.sum(vs[0] + vs[1] + vs[2] + vs[3])
                    totsq = jnp.sum(vs[0] * vs[0] + vs[1] * vs[1]
                                    + vs[2] * vs[2] + vs[3] * vs[3])
                    mean = tot * (1.0 / EMB)
                    var = jnp.maximum(
                        totsq * (1.0 / EMB) - mean * mean, 0.0) + EPS
                    mean_b = jnp.full((LANES,), mean, jnp.float32)
                    rstd_b = _rsqrt(jnp.full((LANES,), var, jnp.float32))
                    for i in range(VPR):
                        out[r, pl.ds(i * LANES, LANES)] = (
                            (vs[i] - mean_b) * rstd_b * g_vecs[i] + b_vecs[i])
                return 0

            lax.fori_loop(0, L // RUNROLL, row_body, 0)

        def store_wait(b):
            pltpu.make_async_copy(out_v[b], out_hbm.at[0], ssem[b]).wait()

        for b in range(NBUF):
            start_gather(b, b)

        def outer(t, _):
            for b in range(NBUF):
                r = t * NBUF + b
                wait_gather(b)

                @pl.when(t >= 1)
                def _():
                    store_wait(b)

                compute_block(b)
                pltpu.async_copy(out_v[b], out_hbm.at[base + r], ssem[b])

                @pl.when(t < n_outer - 1)
                def _():
                    start_gather(r + NBUF, b)
            return 0

        lax.fori_loop(0, n_outer, outer, 0)
        for b in range(NBUF):
            store_wait(b)

    return sc_kernel(idx, table, gamma, beta)
